# Initial kernel scaffold; baseline (speedup 1.0000x reference)
import jax
import jax.numpy as jnp
from jax.experimental import pallas as pl


def kernel(xyz, feature, W1, b1, g1, be1, W2, b2, g2, be2, W3, b3, g3, be3):
    raise NotImplementedError("write your pallas kernel here")


# speedup vs baseline: 5.7576x; 5.7576x over previous
import functools

import jax
import jax.numpy as jnp
from jax import lax
from jax.experimental import pallas as pl
from jax.experimental.pallas import tpu as pltpu
from jax.experimental.pallas import tpu_sc as plsc

B, N, M, K = 16, 2048, 512, 64
C_IN = 128
R2 = 0.2 * 0.2
EPS = 1e-5
CT = 136
P = B * M * K
PB = 512
GROUPS = PB // K
NC, NS = 2, 16
NW = NC * NS
ROWS_PER = (B * M) // NW


def _fps_body(x_ref, y_ref, z_ref, nx_ref, ny_ref, nz_ref):
    iota_n = lax.broadcasted_iota(jnp.int32, (B, N), 1)
    iota_m = lax.broadcasted_iota(jnp.int32, (B, M), 1)
    x = x_ref[...]
    y = y_ref[...]
    z = z_ref[...]

    def step(i, carry):
        mind, far = carry
        eq = iota_n == far
        cx = jnp.sum(jnp.where(eq, x, 0.0), axis=1, keepdims=True)
        cy = jnp.sum(jnp.where(eq, y, 0.0), axis=1, keepdims=True)
        cz = jnp.sum(jnp.where(eq, z, 0.0), axis=1, keepdims=True)
        sel = iota_m == i
        nx_ref[...] = jnp.where(sel, cx, nx_ref[...])
        ny_ref[...] = jnp.where(sel, cy, ny_ref[...])
        nz_ref[...] = jnp.where(sel, cz, nz_ref[...])
        dx = x - cx
        dy = y - cy
        dz = z - cz
        d = dx * dx + dy * dy + dz * dz
        mind = jnp.minimum(mind, d)
        mx = jnp.max(mind, axis=1, keepdims=True)
        far = jnp.min(jnp.where(mind == mx, iota_n, N), axis=1, keepdims=True)
        return mind, far

    init = (jnp.full((B, N), 1e10, jnp.float32), jnp.zeros((B, 1), jnp.int32))
    lax.fori_loop(0, M, step, init)


_fps = pl.pallas_call(
    _fps_body,
    out_shape=[jax.ShapeDtypeStruct((B, M), jnp.float32)] * 3,
)


QB = 128


def _ballq_body(x_ref, y_ref, z_ref, qx_ref, qy_ref, qz_ref, tri_ref,
                pos_ref):
    x = x_ref[0]
    y = y_ref[0]
    z = z_ref[0]
    qx = qx_ref[...]
    qy = qy_ref[...]
    qz = qz_ref[...]

    def tr(v):
        return v.astype(jnp.bfloat16).astype(jnp.float32)

    qp = (tr(qx) * tr(x) + tr(qy) * tr(y)) + tr(qz) * tr(z)
    qq = (qx * qx + qy * qy) + qz * qz
    pp = (x * x + y * y) + z * z
    d2 = qq + pp - 2.0 * qp
    mask = jnp.maximum(jnp.sign(R2 - d2), 0.0).astype(jnp.bfloat16)
    rank = jnp.dot(mask, tri_ref[...], preferred_element_type=jnp.float32)
    c = jnp.minimum(rank, float(K + 1)).astype(jnp.bfloat16)
    ones = jnp.ones((N, 8), jnp.bfloat16)
    iota_k = lax.broadcasted_iota(jnp.int32, (QB, K), 1)

    def kstep(k, acc):
        kf = k.astype(jnp.bfloat16)
        le = jnp.clip(kf - c + 1.0, 0.0, 1.0)
        cnt = jnp.dot(le, ones, preferred_element_type=jnp.float32)
        return jnp.where(iota_k == k, cnt[:, 0:1].astype(jnp.int32), acc)

    pos_ref[...] = lax.fori_loop(
        0, K, kstep, jnp.zeros((QB, K), jnp.int32))


_ballq = pl.pallas_call(
    _ballq_body,
    grid=(B * M // QB,),
    in_specs=[
        pl.BlockSpec((1, 1, N), lambda i: (i // (M // QB), 0, 0)),
        pl.BlockSpec((1, 1, N), lambda i: (i // (M // QB), 0, 0)),
        pl.BlockSpec((1, 1, N), lambda i: (i // (M // QB), 0, 0)),
        pl.BlockSpec((QB, 1), lambda i: (i, 0)),
        pl.BlockSpec((QB, 1), lambda i: (i, 0)),
        pl.BlockSpec((QB, 1), lambda i: (i, 0)),
        pl.BlockSpec((N, N), lambda i: (0, 0)),
    ],
    out_specs=pl.BlockSpec((QB, K), lambda i: (i, 0)),
    out_shape=jax.ShapeDtypeStruct((B * M, K), jnp.int32),
)


def _sc_gather_body(pos_hbm, tbl_hbm, out_hbm, posb, idxv, rows_v, sem):
    cid = lax.axis_index("c")
    sid = lax.axis_index("s")
    wid = sid * NC + cid
    row0 = wid * ROWS_PER
    b = row0 // M
    base_g = b * N
    pltpu.sync_copy(pos_hbm.at[pl.ds(row0 * K, ROWS_PER * K)], posb)

    def row_fn(r, _):
        off = r * K
        f0 = posb[pl.ds(off, 16)][0]
        first = jnp.where(f0 < N, f0, 0)
        for j in range(K // 16):
            v = posb[pl.ds(off + j * 16, 16)]
            v = jnp.where(v < N, v, first)
            idxv[pl.ds(j * 16, 16)] = v + base_g
        pltpu.async_copy(tbl_hbm.at[idxv], rows_v, sem).wait()
        pltpu.sync_copy(rows_v, out_hbm.at[pl.ds((row0 + r) * K, K)])
        return 0

    lax.fori_loop(0, ROWS_PER, row_fn, 0)


@functools.cache
def _sc_gather():
    return pl.kernel(
        _sc_gather_body,
        out_type=jax.ShapeDtypeStruct((P, 128), jnp.float32),
        mesh=plsc.VectorSubcoreMesh(core_axis_name="c", subcore_axis_name="s"),
        scratch_types=[
            pltpu.VMEM((ROWS_PER * K,), jnp.int32),
            pltpu.VMEM((K,), jnp.int32),
            pltpu.VMEM((K, 128), jnp.float32),
            pltpu.SemaphoreType.DMA,
        ],
    )


def _tmm_body(x_ref, w_ref, b_ref, t_ref):
    t_ref[...] = jnp.dot(x_ref[...], w_ref[...],
                         preferred_element_type=jnp.float32) + b_ref[...]


_tmm = pl.pallas_call(
    _tmm_body,
    grid=(B * N // PB,),
    in_specs=[
        pl.BlockSpec((PB, CT), lambda i: (i, 0)),
        pl.BlockSpec((CT, 128), lambda i: (0, 0)),
        pl.BlockSpec((1, 128), lambda i: (0, 0)),
    ],
    out_specs=pl.BlockSpec((PB, 128), lambda i: (i, 0)),
    out_shape=jax.ShapeDtypeStruct((B * N, 128), jnp.float32),
)


def _mm1_body(tg_ref, q_ref, w3_ref, z_ref, ssum_ref, ssq_ref):
    c = jnp.dot(q_ref[...], w3_ref[...], preferred_element_type=jnp.float32)
    z = tg_ref[...]
    z = (z.reshape(GROUPS, K, 128) - c[:, None, :]).reshape(PB, 128)
    z_ref[...] = z

    @pl.when(pl.program_id(0) == 0)
    def _():
        ssum_ref[...] = jnp.zeros_like(ssum_ref)
        ssq_ref[...] = jnp.zeros_like(ssq_ref)

    ssum_ref[...] += jnp.sum(z, axis=0, keepdims=True)
    ssq_ref[...] += jnp.sum(z * z, axis=0, keepdims=True)


def _mm_body(z_in_ref, sc_ref, sh_ref, w_ref, b_ref, z_ref, ssum_ref, ssq_ref):
    h = jnp.maximum(z_in_ref[...] * sc_ref[...] + sh_ref[...], 0.0)
    z = jnp.dot(h, w_ref[...], preferred_element_type=jnp.float32) + b_ref[...]
    z_ref[...] = z

    @pl.when(pl.program_id(0) == 0)
    def _():
        ssum_ref[...] = jnp.zeros_like(ssum_ref)
        ssq_ref[...] = jnp.zeros_like(ssq_ref)

    ssum_ref[...] += jnp.sum(z, axis=0, keepdims=True)
    ssq_ref[...] += jnp.sum(z * z, axis=0, keepdims=True)


def _pool_body(z_ref, sc_ref, sh_ref, o_ref):
    h = jnp.maximum(z_ref[...] * sc_ref[...] + sh_ref[...], 0.0)
    o_ref[...] = jnp.max(h.reshape(GROUPS, K, 256), axis=1)


def _stats_block(co):
    return [
        pl.BlockSpec((PB, co), lambda i: (i, 0)),
        pl.BlockSpec((1, co), lambda i: (0, 0)),
        pl.BlockSpec((1, co), lambda i: (0, 0)),
    ]


_mm1 = pl.pallas_call(
    _mm1_body,
    grid=(P // PB,),
    in_specs=[
        pl.BlockSpec((PB, 128), lambda i: (i, 0)),
        pl.BlockSpec((GROUPS, 3), lambda i: (i, 0)),
        pl.BlockSpec((3, 128), lambda i: (0, 0)),
    ],
    out_specs=_stats_block(128),
    out_shape=[
        jax.ShapeDtypeStruct((P, 128), jnp.float32),
        jax.ShapeDtypeStruct((1, 128), jnp.float32),
        jax.ShapeDtypeStruct((1, 128), jnp.float32),
    ],
)


def _make_mm(ci, co):
    return pl.pallas_call(
        _mm_body,
        grid=(P // PB,),
        in_specs=[
            pl.BlockSpec((PB, ci), lambda i: (i, 0)),
            pl.BlockSpec((1, ci), lambda i: (0, 0)),
            pl.BlockSpec((1, ci), lambda i: (0, 0)),
            pl.BlockSpec((ci, co), lambda i: (0, 0)),
            pl.BlockSpec((1, co), lambda i: (0, 0)),
        ],
        out_specs=_stats_block(co),
        out_shape=[
            jax.ShapeDtypeStruct((P, co), jnp.float32),
            jax.ShapeDtypeStruct((1, co), jnp.float32),
            jax.ShapeDtypeStruct((1, co), jnp.float32),
        ],
    )


_mm2 = _make_mm(128, 128)
_mm3 = _make_mm(128, 256)

_pool = pl.pallas_call(
    _pool_body,
    grid=(P // PB,),
    in_specs=[
        pl.BlockSpec((PB, 256), lambda i: (i, 0)),
        pl.BlockSpec((1, 256), lambda i: (0, 0)),
        pl.BlockSpec((1, 256), lambda i: (0, 0)),
    ],
    out_specs=pl.BlockSpec((GROUPS, 256), lambda i: (i, 0)),
    out_shape=jax.ShapeDtypeStruct((B * M, 256), jnp.float32),
)


def _affine(ssum, ssq, g, be):
    mean = ssum[0] / P
    var = ssq[0] / P - mean * mean
    scale = g / jnp.sqrt(var + EPS)
    shift = be - mean * scale
    return scale[None, :], shift[None, :]


@jax.jit
def kernel(xyz, feature, W1, b1, g1, be1, W2, b2, g2, be2, W3, b3, g3, be3):
    x = xyz[:, 0, :]
    y = xyz[:, 1, :]
    z = xyz[:, 2, :]
    nx, ny, nz = _fps(x, y, z)
    new_xyz = jnp.stack([nx, ny, nz], axis=1)

    tbl = jnp.concatenate(
        [jnp.transpose(xyz, (0, 2, 1)),
         jnp.transpose(feature, (0, 2, 1)),
         jnp.zeros((B, N, CT - 3 - C_IN), jnp.float32)], axis=-1,
    ).reshape(B * N, CT)

    qx = nx.reshape(B * M, 1)
    qy = ny.reshape(B * M, 1)
    qz = nz.reshape(B * M, 1)
    tri = jnp.triu(jnp.ones((N, N), jnp.bfloat16))
    x3 = x.reshape(B, 1, N)
    y3 = y.reshape(B, 1, N)
    z3 = z.reshape(B, 1, N)
    pos = _ballq(x3, y3, z3, qx, qy, qz, tri)

    w1t = jnp.zeros((CT, 128), jnp.float32).at[:3 + C_IN, :].set(W1.T)
    t = _tmm(tbl, w1t, b1[None, :])
    tg = _sc_gather()(pos.reshape(B * M * K), t)
    q = jnp.stack([nx, ny, nz], axis=-1).reshape(B * M, 3)

    z1, s1, q1 = _mm1(tg, q, W1[:, :3].T)
    sc1, sh1 = _affine(s1, q1, g1, be1)
    z2, s2, q2 = _mm2(z1, sc1, sh1, W2.T, b2[None, :])
    sc2, sh2 = _affine(s2, q2, g2, be2)
    z3, s3, q3 = _mm3(z2, sc2, sh2, W3.T, b3[None, :])
    sc3, sh3 = _affine(s3, q3, g3, be3)
    pooled = _pool(z3, sc3, sh3)
    nf = jnp.transpose(pooled.reshape(B, M, 256), (0, 2, 1))
    return (new_xyz, nf)


# speedup vs baseline: 5.9664x; 1.0363x over previous
import functools

import jax
import jax.numpy as jnp
from jax import lax
from jax.experimental import pallas as pl
from jax.experimental.pallas import tpu as pltpu
from jax.experimental.pallas import tpu_sc as plsc

B, N, M, K = 16, 2048, 512, 64
C_IN = 128
R2 = 0.2 * 0.2
EPS = 1e-5
CT = 136
P = B * M * K
PB = 512
GROUPS = PB // K
NC, NS = 2, 16
NW = NC * NS
ROWS_PER = (B * M) // NW


def _fps_body(x_ref, y_ref, z_ref, nx_ref, ny_ref, nz_ref):
    iota_n = lax.broadcasted_iota(jnp.int32, (B, N), 1)
    iota_m = lax.broadcasted_iota(jnp.int32, (B, M), 1)
    x = x_ref[...]
    y = y_ref[...]
    z = z_ref[...]

    def step(i, carry):
        mind, far = carry
        eq = iota_n == far
        cx = jnp.sum(jnp.where(eq, x, 0.0), axis=1, keepdims=True)
        cy = jnp.sum(jnp.where(eq, y, 0.0), axis=1, keepdims=True)
        cz = jnp.sum(jnp.where(eq, z, 0.0), axis=1, keepdims=True)
        sel = iota_m == i
        nx_ref[...] = jnp.where(sel, cx, nx_ref[...])
        ny_ref[...] = jnp.where(sel, cy, ny_ref[...])
        nz_ref[...] = jnp.where(sel, cz, nz_ref[...])
        dx = x - cx
        dy = y - cy
        dz = z - cz
        d = dx * dx + dy * dy + dz * dz
        mind = jnp.minimum(mind, d)
        mx = jnp.max(mind, axis=1, keepdims=True)
        far = jnp.min(jnp.where(mind == mx, iota_n, N), axis=1, keepdims=True)
        return mind, far

    init = (jnp.full((B, N), 1e10, jnp.float32), jnp.zeros((B, 1), jnp.int32))
    lax.fori_loop(0, M, step, init)


_fps = pl.pallas_call(
    _fps_body,
    out_shape=[jax.ShapeDtypeStruct((B, M), jnp.float32)] * 3,
)


QB = 128


def _ballq_body(x_ref, y_ref, z_ref, qx_ref, qy_ref, qz_ref, tri_ref,
                pos_ref):
    x = x_ref[0]
    y = y_ref[0]
    z = z_ref[0]
    qx = qx_ref[...]
    qy = qy_ref[...]
    qz = qz_ref[...]

    def tr(v):
        return v.astype(jnp.bfloat16).astype(jnp.float32)

    qp = (tr(qx) * tr(x) + tr(qy) * tr(y)) + tr(qz) * tr(z)
    qq = (qx * qx + qy * qy) + qz * qz
    pp = (x * x + y * y) + z * z
    d2 = qq + pp - 2.0 * qp
    mask = jnp.maximum(jnp.sign(R2 - d2), 0.0).astype(jnp.bfloat16)
    rank = jnp.dot(mask, tri_ref[...], preferred_element_type=jnp.float32)
    c = jnp.minimum(rank, float(K + 1)).astype(jnp.bfloat16)
    ones = jnp.ones((N, 8), jnp.bfloat16)
    iota_k = lax.broadcasted_iota(jnp.int32, (QB, K), 1)

    def kstep(k, acc):
        kf = k.astype(jnp.bfloat16)
        le = jnp.clip(kf - c + 1.0, 0.0, 1.0)
        cnt = jnp.dot(le, ones, preferred_element_type=jnp.float32)
        return jnp.where(iota_k == k, cnt[:, 0:1].astype(jnp.int32), acc)

    pos_ref[...] = lax.fori_loop(
        0, K, kstep, jnp.zeros((QB, K), jnp.int32))


_ballq = pl.pallas_call(
    _ballq_body,
    grid=(B * M // QB,),
    in_specs=[
        pl.BlockSpec((1, 1, N), lambda i: (i // (M // QB), 0, 0)),
        pl.BlockSpec((1, 1, N), lambda i: (i // (M // QB), 0, 0)),
        pl.BlockSpec((1, 1, N), lambda i: (i // (M // QB), 0, 0)),
        pl.BlockSpec((QB, 1), lambda i: (i, 0)),
        pl.BlockSpec((QB, 1), lambda i: (i, 0)),
        pl.BlockSpec((QB, 1), lambda i: (i, 0)),
        pl.BlockSpec((N, N), lambda i: (0, 0)),
    ],
    out_specs=pl.BlockSpec((QB, K), lambda i: (i, 0)),
    out_shape=jax.ShapeDtypeStruct((B * M, K), jnp.int32),
)


def _sc_gather_body(pos_hbm, tbl_hbm, out_hbm, posb, idxv, rows_v, sem):
    cid = lax.axis_index("c")
    sid = lax.axis_index("s")
    wid = sid * NC + cid
    row0 = wid * ROWS_PER
    b = row0 // M
    base_g = b * N
    pltpu.sync_copy(pos_hbm.at[pl.ds(row0 * K, ROWS_PER * K)], posb)

    def pair_fn(p, _):
        for h in range(2):
            off = (2 * p + h) * K
            f0 = posb[pl.ds(off, 16)][0]
            first = jnp.where(f0 < N, f0, 0)
            for j in range(K // 16):
                v = posb[pl.ds(off + j * 16, 16)]
                v = jnp.where(v < N, v, first)
                idxv[pl.ds(h * K + j * 16, 16)] = v + base_g
        pltpu.async_copy(tbl_hbm.at[idxv], rows_v, sem).wait()
        pltpu.sync_copy(rows_v,
                        out_hbm.at[pl.ds((row0 + 2 * p) * K, 2 * K)])
        return 0

    lax.fori_loop(0, ROWS_PER // 2, pair_fn, 0)


@functools.cache
def _sc_gather():
    return pl.kernel(
        _sc_gather_body,
        out_type=jax.ShapeDtypeStruct((P, 128), jnp.float32),
        mesh=plsc.VectorSubcoreMesh(core_axis_name="c", subcore_axis_name="s"),
        scratch_types=[
            pltpu.VMEM((ROWS_PER * K,), jnp.int32),
            pltpu.VMEM((2 * K,), jnp.int32),
            pltpu.VMEM((2 * K, 128), jnp.float32),
            pltpu.SemaphoreType.DMA,
        ],
    )


def _tmm_body(x_ref, w_ref, b_ref, t_ref):
    t_ref[...] = jnp.dot(x_ref[...], w_ref[...],
                         preferred_element_type=jnp.float32) + b_ref[...]


_tmm = pl.pallas_call(
    _tmm_body,
    grid=(B * N // PB,),
    in_specs=[
        pl.BlockSpec((PB, CT), lambda i: (i, 0)),
        pl.BlockSpec((CT, 128), lambda i: (0, 0)),
        pl.BlockSpec((1, 128), lambda i: (0, 0)),
    ],
    out_specs=pl.BlockSpec((PB, 128), lambda i: (i, 0)),
    out_shape=jax.ShapeDtypeStruct((B * N, 128), jnp.float32),
)


def _mm1_body(tg_ref, q_ref, w3_ref, z_ref, ssum_ref, ssq_ref):
    c = jnp.dot(q_ref[...], w3_ref[...], preferred_element_type=jnp.float32)
    z = tg_ref[...]
    z = (z.reshape(GROUPS, K, 128) - c[:, None, :]).reshape(PB, 128)
    z_ref[...] = z

    @pl.when(pl.program_id(0) == 0)
    def _():
        ssum_ref[...] = jnp.zeros_like(ssum_ref)
        ssq_ref[...] = jnp.zeros_like(ssq_ref)

    ssum_ref[...] += jnp.sum(z, axis=0, keepdims=True)
    ssq_ref[...] += jnp.sum(z * z, axis=0, keepdims=True)


def _mm_body(z_in_ref, sc_ref, sh_ref, w_ref, b_ref, z_ref, ssum_ref, ssq_ref):
    h = jnp.maximum(z_in_ref[...] * sc_ref[...] + sh_ref[...], 0.0)
    z = jnp.dot(h, w_ref[...], preferred_element_type=jnp.float32) + b_ref[...]
    z_ref[...] = z

    @pl.when(pl.program_id(0) == 0)
    def _():
        ssum_ref[...] = jnp.zeros_like(ssum_ref)
        ssq_ref[...] = jnp.zeros_like(ssq_ref)

    ssum_ref[...] += jnp.sum(z, axis=0, keepdims=True)
    ssq_ref[...] += jnp.sum(z * z, axis=0, keepdims=True)


def _pool_body(z_ref, sc_ref, sh_ref, o_ref):
    h = jnp.maximum(z_ref[...] * sc_ref[...] + sh_ref[...], 0.0)
    o_ref[...] = jnp.max(h.reshape(GROUPS, K, 256), axis=1)


def _stats_block(co):
    return [
        pl.BlockSpec((PB, co), lambda i: (i, 0)),
        pl.BlockSpec((1, co), lambda i: (0, 0)),
        pl.BlockSpec((1, co), lambda i: (0, 0)),
    ]


_mm1 = pl.pallas_call(
    _mm1_body,
    grid=(P // PB,),
    in_specs=[
        pl.BlockSpec((PB, 128), lambda i: (i, 0)),
        pl.BlockSpec((GROUPS, 3), lambda i: (i, 0)),
        pl.BlockSpec((3, 128), lambda i: (0, 0)),
    ],
    out_specs=_stats_block(128),
    out_shape=[
        jax.ShapeDtypeStruct((P, 128), jnp.float32),
        jax.ShapeDtypeStruct((1, 128), jnp.float32),
        jax.ShapeDtypeStruct((1, 128), jnp.float32),
    ],
)


def _make_mm(ci, co):
    return pl.pallas_call(
        _mm_body,
        grid=(P // PB,),
        in_specs=[
            pl.BlockSpec((PB, ci), lambda i: (i, 0)),
            pl.BlockSpec((1, ci), lambda i: (0, 0)),
            pl.BlockSpec((1, ci), lambda i: (0, 0)),
            pl.BlockSpec((ci, co), lambda i: (0, 0)),
            pl.BlockSpec((1, co), lambda i: (0, 0)),
        ],
        out_specs=_stats_block(co),
        out_shape=[
            jax.ShapeDtypeStruct((P, co), jnp.float32),
            jax.ShapeDtypeStruct((1, co), jnp.float32),
            jax.ShapeDtypeStruct((1, co), jnp.float32),
        ],
    )


_mm2 = _make_mm(128, 128)
_mm3 = _make_mm(128, 256)

_pool = pl.pallas_call(
    _pool_body,
    grid=(P // PB,),
    in_specs=[
        pl.BlockSpec((PB, 256), lambda i: (i, 0)),
        pl.BlockSpec((1, 256), lambda i: (0, 0)),
        pl.BlockSpec((1, 256), lambda i: (0, 0)),
    ],
    out_specs=pl.BlockSpec((GROUPS, 256), lambda i: (i, 0)),
    out_shape=jax.ShapeDtypeStruct((B * M, 256), jnp.float32),
)


def _affine(ssum, ssq, g, be):
    mean = ssum[0] / P
    var = ssq[0] / P - mean * mean
    scale = g / jnp.sqrt(var + EPS)
    shift = be - mean * scale
    return scale[None, :], shift[None, :]


@jax.jit
def kernel(xyz, feature, W1, b1, g1, be1, W2, b2, g2, be2, W3, b3, g3, be3):
    x = xyz[:, 0, :]
    y = xyz[:, 1, :]
    z = xyz[:, 2, :]
    nx, ny, nz = _fps(x, y, z)
    new_xyz = jnp.stack([nx, ny, nz], axis=1)

    tbl = jnp.concatenate(
        [jnp.transpose(xyz, (0, 2, 1)),
         jnp.transpose(feature, (0, 2, 1)),
         jnp.zeros((B, N, CT - 3 - C_IN), jnp.float32)], axis=-1,
    ).reshape(B * N, CT)

    qx = nx.reshape(B * M, 1)
    qy = ny.reshape(B * M, 1)
    qz = nz.reshape(B * M, 1)
    tri = jnp.triu(jnp.ones((N, N), jnp.bfloat16))
    x3 = x.reshape(B, 1, N)
    y3 = y.reshape(B, 1, N)
    z3 = z.reshape(B, 1, N)
    pos = _ballq(x3, y3, z3, qx, qy, qz, tri)

    w1t = jnp.zeros((CT, 128), jnp.float32).at[:3 + C_IN, :].set(W1.T)
    t = _tmm(tbl, w1t, b1[None, :])
    tg = _sc_gather()(pos.reshape(B * M * K), t)
    q = jnp.stack([nx, ny, nz], axis=-1).reshape(B * M, 3)

    z1, s1, q1 = _mm1(tg, q, W1[:, :3].T)
    sc1, sh1 = _affine(s1, q1, g1, be1)
    z2, s2, q2 = _mm2(z1, sc1, sh1, W2.T, b2[None, :])
    sc2, sh2 = _affine(s2, q2, g2, be2)
    z3, s3, q3 = _mm3(z2, sc2, sh2, W3.T, b3[None, :])
    sc3, sh3 = _affine(s3, q3, g3, be3)
    pooled = _pool(z3, sc3, sh3)
    nf = jnp.transpose(pooled.reshape(B, M, 256), (0, 2, 1))
    return (new_xyz, nf)


# speedup vs baseline: 6.2048x; 1.0399x over previous
import functools

import jax
import jax.numpy as jnp
from jax import lax
from jax.experimental import pallas as pl
from jax.experimental.pallas import tpu as pltpu
from jax.experimental.pallas import tpu_sc as plsc

B, N, M, K = 16, 2048, 512, 64
C_IN = 128
R2 = 0.2 * 0.2
EPS = 1e-5
CT = 136
P = B * M * K
PB = 512
GROUPS = PB // K
NC, NS = 2, 16
NW = NC * NS
ROWS_PER = (B * M) // NW


def _fps_body(x_ref, y_ref, z_ref, nx_ref, ny_ref, nz_ref):
    iota_n = lax.broadcasted_iota(jnp.int32, (B, N), 1)
    iota_m = lax.broadcasted_iota(jnp.int32, (B, M), 1)
    x = x_ref[...]
    y = y_ref[...]
    z = z_ref[...]

    def step(i, carry):
        mind, far = carry
        eq = iota_n == far
        cx = jnp.sum(jnp.where(eq, x, 0.0), axis=1, keepdims=True)
        cy = jnp.sum(jnp.where(eq, y, 0.0), axis=1, keepdims=True)
        cz = jnp.sum(jnp.where(eq, z, 0.0), axis=1, keepdims=True)
        sel = iota_m == i
        nx_ref[...] = jnp.where(sel, cx, nx_ref[...])
        ny_ref[...] = jnp.where(sel, cy, ny_ref[...])
        nz_ref[...] = jnp.where(sel, cz, nz_ref[...])
        dx = x - cx
        dy = y - cy
        dz = z - cz
        d = dx * dx + dy * dy + dz * dz
        mind = jnp.minimum(mind, d)
        mx = jnp.max(mind, axis=1, keepdims=True)
        far = jnp.min(jnp.where(mind == mx, iota_n, N), axis=1, keepdims=True)
        return mind, far

    init = (jnp.full((B, N), 1e10, jnp.float32), jnp.zeros((B, 1), jnp.int32))
    lax.fori_loop(0, M, step, init)


_fps = pl.pallas_call(
    _fps_body,
    out_shape=[jax.ShapeDtypeStruct((B, M), jnp.float32)] * 3,
)


QB = 128


def _ballq_body(x_ref, y_ref, z_ref, qx_ref, qy_ref, qz_ref, tri_ref,
                pos_ref):
    x = x_ref[0]
    y = y_ref[0]
    z = z_ref[0]
    qx = qx_ref[...]
    qy = qy_ref[...]
    qz = qz_ref[...]

    def tr(v):
        return v.astype(jnp.bfloat16).astype(jnp.float32)

    qp = (tr(qx) * tr(x) + tr(qy) * tr(y)) + tr(qz) * tr(z)
    qq = (qx * qx + qy * qy) + qz * qz
    pp = (x * x + y * y) + z * z
    d2 = qq + pp - 2.0 * qp
    mask = jnp.maximum(jnp.sign(R2 - d2), 0.0).astype(jnp.bfloat16)
    rank = jnp.dot(mask, tri_ref[...], preferred_element_type=jnp.float32)
    c = jnp.minimum(rank, float(K + 1)).astype(jnp.bfloat16)
    ones = jnp.ones((N, 8), jnp.bfloat16)
    iota_k = lax.broadcasted_iota(jnp.int32, (QB, K), 1)

    def kstep(k, acc):
        kf = k.astype(jnp.bfloat16)
        le = jnp.clip(kf - c + 1.0, 0.0, 1.0)
        cnt = jnp.dot(le, ones, preferred_element_type=jnp.float32)
        return jnp.where(iota_k == k, cnt[:, 0:1].astype(jnp.int32), acc)

    pos_ref[...] = lax.fori_loop(
        0, K, kstep, jnp.zeros((QB, K), jnp.int32))


_ballq = pl.pallas_call(
    _ballq_body,
    grid=(B * M // QB,),
    in_specs=[
        pl.BlockSpec((1, 1, N), lambda i: (i // (M // QB), 0, 0)),
        pl.BlockSpec((1, 1, N), lambda i: (i // (M // QB), 0, 0)),
        pl.BlockSpec((1, 1, N), lambda i: (i // (M // QB), 0, 0)),
        pl.BlockSpec((QB, 1), lambda i: (i, 0)),
        pl.BlockSpec((QB, 1), lambda i: (i, 0)),
        pl.BlockSpec((QB, 1), lambda i: (i, 0)),
        pl.BlockSpec((N, N), lambda i: (0, 0)),
    ],
    out_specs=pl.BlockSpec((QB, K), lambda i: (i, 0)),
    out_shape=jax.ShapeDtypeStruct((B * M, K), jnp.int32),
)


def _sc_gather_body(pos_hbm, tbl_hbm, out_hbm, posb, idxv, rows_v, sem):
    cid = lax.axis_index("c")
    sid = lax.axis_index("s")
    wid = sid * NC + cid
    row0 = wid * ROWS_PER
    b = row0 // M
    base_g = b * N
    pltpu.sync_copy(pos_hbm.at[pl.ds(row0 * K, ROWS_PER * K)], posb)

    def pair_fn(p, _):
        for h in range(2):
            off = (2 * p + h) * K
            f0 = posb[pl.ds(off, 16)][0]
            first = jnp.where(f0 < N, f0, 0)
            for j in range(K // 16):
                v = posb[pl.ds(off + j * 16, 16)]
                v = jnp.where(v < N, v, first)
                idxv[pl.ds(h * K + j * 16, 16)] = v + base_g
        pltpu.async_copy(tbl_hbm.at[idxv], rows_v, sem).wait()
        pltpu.sync_copy(rows_v,
                        out_hbm.at[pl.ds((row0 + 2 * p) * K, 2 * K)])
        return 0

    lax.fori_loop(0, ROWS_PER // 2, pair_fn, 0)


@functools.cache
def _sc_gather():
    return pl.kernel(
        _sc_gather_body,
        out_type=jax.ShapeDtypeStruct((P, 128), jnp.float32),
        mesh=plsc.VectorSubcoreMesh(core_axis_name="c", subcore_axis_name="s"),
        scratch_types=[
            pltpu.VMEM((ROWS_PER * K,), jnp.int32),
            pltpu.VMEM((2 * K,), jnp.int32),
            pltpu.VMEM((2 * K, 128), jnp.float32),
            pltpu.SemaphoreType.DMA,
        ],
    )


def _tmm_body(x_ref, w_ref, b_ref, t_ref):
    t_ref[...] = jnp.dot(x_ref[...], w_ref[...],
                         preferred_element_type=jnp.float32) + b_ref[...]


_tmm = pl.pallas_call(
    _tmm_body,
    grid=(B * N // PB,),
    in_specs=[
        pl.BlockSpec((PB, CT), lambda i: (i, 0)),
        pl.BlockSpec((CT, 128), lambda i: (0, 0)),
        pl.BlockSpec((1, 128), lambda i: (0, 0)),
    ],
    out_specs=pl.BlockSpec((PB, 128), lambda i: (i, 0)),
    out_shape=jax.ShapeDtypeStruct((B * N, 128), jnp.float32),
)


def _mm1_body(tg_ref, q_ref, w3_ref, z_ref, ssum_ref, ssq_ref):
    c = jnp.dot(q_ref[...], w3_ref[...], preferred_element_type=jnp.float32)
    z = tg_ref[...]
    z = (z.reshape(GROUPS, K, 128) - c[:, None, :]).reshape(PB, 128)
    z_ref[...] = z.astype(jnp.bfloat16)

    @pl.when(pl.program_id(0) == 0)
    def _():
        ssum_ref[...] = jnp.zeros_like(ssum_ref)
        ssq_ref[...] = jnp.zeros_like(ssq_ref)

    ssum_ref[...] += jnp.sum(z, axis=0, keepdims=True)
    ssq_ref[...] += jnp.sum(z * z, axis=0, keepdims=True)


def _mm_body(z_in_ref, sc_ref, sh_ref, w_ref, b_ref, z_ref, ssum_ref, ssq_ref):
    zin = z_in_ref[...].astype(jnp.float32)
    h = jnp.maximum(zin * sc_ref[...] + sh_ref[...], 0.0)
    z = jnp.dot(h.astype(jnp.bfloat16), w_ref[...],
                preferred_element_type=jnp.float32) + b_ref[...]
    z_ref[...] = z.astype(jnp.bfloat16)

    @pl.when(pl.program_id(0) == 0)
    def _():
        ssum_ref[...] = jnp.zeros_like(ssum_ref)
        ssq_ref[...] = jnp.zeros_like(ssq_ref)

    ssum_ref[...] += jnp.sum(z, axis=0, keepdims=True)
    ssq_ref[...] += jnp.sum(z * z, axis=0, keepdims=True)


def _pool_body(z_ref, sc_ref, sh_ref, o_ref):
    z = z_ref[...].astype(jnp.float32)
    h = jnp.maximum(z * sc_ref[...] + sh_ref[...], 0.0)
    o_ref[...] = jnp.max(h.reshape(GROUPS, K, 256), axis=1)


def _stats_block(co):
    return [
        pl.BlockSpec((PB, co), lambda i: (i, 0)),
        pl.BlockSpec((1, co), lambda i: (0, 0)),
        pl.BlockSpec((1, co), lambda i: (0, 0)),
    ]


_mm1 = pl.pallas_call(
    _mm1_body,
    grid=(P // PB,),
    in_specs=[
        pl.BlockSpec((PB, 128), lambda i: (i, 0)),
        pl.BlockSpec((GROUPS, 3), lambda i: (i, 0)),
        pl.BlockSpec((3, 128), lambda i: (0, 0)),
    ],
    out_specs=_stats_block(128),
    out_shape=[
        jax.ShapeDtypeStruct((P, 128), jnp.bfloat16),
        jax.ShapeDtypeStruct((1, 128), jnp.float32),
        jax.ShapeDtypeStruct((1, 128), jnp.float32),
    ],
)


def _make_mm(ci, co):
    return pl.pallas_call(
        _mm_body,
        grid=(P // PB,),
        in_specs=[
            pl.BlockSpec((PB, ci), lambda i: (i, 0)),
            pl.BlockSpec((1, ci), lambda i: (0, 0)),
            pl.BlockSpec((1, ci), lambda i: (0, 0)),
            pl.BlockSpec((ci, co), lambda i: (0, 0)),
            pl.BlockSpec((1, co), lambda i: (0, 0)),
        ],
        out_specs=_stats_block(co),
        out_shape=[
            jax.ShapeDtypeStruct((P, co), jnp.bfloat16),
            jax.ShapeDtypeStruct((1, co), jnp.float32),
            jax.ShapeDtypeStruct((1, co), jnp.float32),
        ],
    )


_mm2 = _make_mm(128, 128)
_mm3 = _make_mm(128, 256)

_pool = pl.pallas_call(
    _pool_body,
    grid=(P // PB,),
    in_specs=[
        pl.BlockSpec((PB, 256), lambda i: (i, 0)),
        pl.BlockSpec((1, 256), lambda i: (0, 0)),
        pl.BlockSpec((1, 256), lambda i: (0, 0)),
    ],
    out_specs=pl.BlockSpec((GROUPS, 256), lambda i: (i, 0)),
    out_shape=jax.ShapeDtypeStruct((B * M, 256), jnp.float32),
)


def _affine(ssum, ssq, g, be):
    mean = ssum[0] / P
    var = ssq[0] / P - mean * mean
    scale = g / jnp.sqrt(var + EPS)
    shift = be - mean * scale
    return scale[None, :], shift[None, :]


@jax.jit
def kernel(xyz, feature, W1, b1, g1, be1, W2, b2, g2, be2, W3, b3, g3, be3):
    x = xyz[:, 0, :]
    y = xyz[:, 1, :]
    z = xyz[:, 2, :]
    nx, ny, nz = _fps(x, y, z)
    new_xyz = jnp.stack([nx, ny, nz], axis=1)

    tbl = jnp.concatenate(
        [jnp.transpose(xyz, (0, 2, 1)),
         jnp.transpose(feature, (0, 2, 1)),
         jnp.zeros((B, N, CT - 3 - C_IN), jnp.float32)], axis=-1,
    ).reshape(B * N, CT)

    qx = nx.reshape(B * M, 1)
    qy = ny.reshape(B * M, 1)
    qz = nz.reshape(B * M, 1)
    tri = jnp.triu(jnp.ones((N, N), jnp.bfloat16))
    x3 = x.reshape(B, 1, N)
    y3 = y.reshape(B, 1, N)
    z3 = z.reshape(B, 1, N)
    pos = _ballq(x3, y3, z3, qx, qy, qz, tri)

    w1t = jnp.zeros((CT, 128), jnp.float32).at[:3 + C_IN, :].set(W1.T)
    t = _tmm(tbl, w1t, b1[None, :])
    tg = _sc_gather()(pos.reshape(B * M * K), t)
    q = jnp.stack([nx, ny, nz], axis=-1).reshape(B * M, 3)

    z1, s1, q1 = _mm1(tg, q, W1[:, :3].T)
    sc1, sh1 = _affine(s1, q1, g1, be1)
    z2, s2, q2 = _mm2(z1, sc1, sh1, W2.T.astype(jnp.bfloat16), b2[None, :])
    sc2, sh2 = _affine(s2, q2, g2, be2)
    z3, s3, q3 = _mm3(z2, sc2, sh2, W3.T.astype(jnp.bfloat16), b3[None, :])
    sc3, sh3 = _affine(s3, q3, g3, be3)
    pooled = _pool(z3, sc3, sh3)
    nf = jnp.transpose(pooled.reshape(B, M, 256), (0, 2, 1))
    return (new_xyz, nf)
